# baseline (speedup 1.0000x reference)
import functools

import jax
import jax.numpy as jnp
from jax import lax
from jax.experimental import pallas as pl
from jax.experimental.pallas import tpu as pltpu
from jax.experimental.pallas import tpu_sc as plsc

N = 10000
D = 128
E = 320000

NC = 2
NS = 16
NW = NC * NS
CHUNK = 40
NBUF = 5
EPW = E // NW
NCHUNK = EPW // CHUNK
NGROUP = NCHUNK // NBUF
TOTCHUNK = E // CHUNK

NP_ROWS = 10240
RPT = NP_ROWS // NS

DEG_FIRE = 25


def _sc_degree(edge4d):
    mesh = plsc.VectorSubcoreMesh(core_axis_name="c", subcore_axis_name="s")

    @functools.partial(
        pl.kernel,
        out_type=jax.ShapeDtypeStruct((NC, NP_ROWS), jnp.float32),
        mesh=mesh,
        scratch_types=[
            pltpu.VMEM((NCHUNK, CHUNK), jnp.int32),
            pltpu.VMEM((48,), jnp.float32),
            pltpu.VMEM((RPT,), jnp.float32),
            pltpu.VMEM_SHARED((NP_ROWS,), jnp.float32),
            pltpu.SemaphoreType.DMA,
        ],
    )
    def k(edge_hbm, deg_out, idx_v, ones_v, zero_v, deg_sh, sem):
        cid = lax.axis_index("c")
        sid = lax.axis_index("s")
        wid = sid * NC + cid
        pltpu.sync_copy(edge_hbm.at[1, wid], idx_v)

        for j in range(48 // 16):
            ones_v[pl.ds(j * 16, 16)] = jnp.ones((16,), jnp.float32)
        ones_c = ones_v.at[pl.ds(0, CHUNK)]

        def zfill(i, carry):
            zero_v[pl.ds(i * 16, 16)] = jnp.zeros((16,), jnp.float32)
            return carry

        lax.fori_loop(0, RPT // 16, zfill, 0)
        pltpu.sync_copy(zero_v, deg_sh.at[pl.ds(sid * RPT, RPT)])
        plsc.subcore_barrier()

        def body(grp, carry):
            for j in range(DEG_FIRE):
                c = grp * DEG_FIRE + j
                pltpu.async_copy(ones_c, deg_sh.at[idx_v.at[c]], sem,
                                 add=True)
            for j in range(DEG_FIRE):
                c = grp * DEG_FIRE + j
                pltpu.make_async_copy(ones_c, deg_sh.at[idx_v.at[c]],
                                      sem).wait()
            return carry

        lax.fori_loop(0, NCHUNK // DEG_FIRE, body, 0)
        plsc.subcore_barrier()
        pltpu.sync_copy(deg_sh.at[pl.ds(sid * RPT, RPT)],
                        deg_out.at[cid, pl.ds(sid * RPT, RPT)])

    return k(edge4d)


def _tc_linear(x, wt, degp):
    BR = 2000

    def body(x_ref, wt_ref, deg_ref, o_ref):
        degb = deg_ref[...]
        deg = degb[:, 0:1] + degb[:, 1:2]
        dinv = jnp.where(deg > 0, lax.rsqrt(deg), 0.0)
        o_ref[...] = jnp.dot(x_ref[...] * dinv, wt_ref[...],
                             preferred_element_type=jnp.float32)

    return pl.pallas_call(
        body,
        grid=(N // BR,),
        in_specs=[
            pl.BlockSpec((BR, D), lambda i: (i, 0)),
            pl.BlockSpec((D, D), lambda i: (0, 0)),
            pl.BlockSpec((BR, 2), lambda i: (i, 0)),
        ],
        out_specs=pl.BlockSpec((BR, D), lambda i: (i, 0)),
        out_shape=jax.ShapeDtypeStruct((N, D), jnp.float32),
    )(x, wt, degp)


def _sc_gather_scatter(h, edge5d):
    mesh = plsc.VectorSubcoreMesh(core_axis_name="c", subcore_axis_name="s")

    @functools.partial(
        pl.kernel,
        out_type=jax.ShapeDtypeStruct((NC, NP_ROWS, D), jnp.float32),
        mesh=mesh,
        scratch_types=[
            [pltpu.VMEM((CHUNK, D), jnp.float32) for _ in range(NBUF)],
            [pltpu.SemaphoreType.DMA for _ in range(NBUF)],
            [pltpu.VMEM((NBUF, CHUNK), jnp.int32) for _ in range(2)],
            [pltpu.VMEM((NBUF, CHUNK), jnp.int32) for _ in range(2)],
            pltpu.SemaphoreType.DMA,
            pltpu.VMEM_SHARED((NP_ROWS, D), jnp.float32),
        ],
    )
    def k(h_hbm, edge_hbm, out_hbm, bufs, gsem, rsb, csb, ssem, acc_sh):
        cid = lax.axis_index("c")
        sid = lax.axis_index("s")
        wid = sid * NC + cid
        row0 = sid * RPT

        def zfill(i, carry):
            for j in range(D // 16):
                bufs[0][i, pl.ds(j * 16, 16)] = jnp.zeros((16,), jnp.float32)
            return carry

        lax.fori_loop(0, CHUNK, zfill, 0)
        for j in range(RPT // CHUNK):
            pltpu.sync_copy(bufs[0], acc_sh.at[pl.ds(row0 + j * CHUNK, CHUNK)])
        plsc.subcore_barrier()

        def slab_load(g, r, c_, sem):
            pltpu.async_copy(edge_hbm.at[0, wid, g], r, sem)
            pltpu.async_copy(edge_hbm.at[1, wid, g], c_, sem)

        def slab_wait(g, r, c_, sem):
            pltpu.make_async_copy(edge_hbm.at[0, wid, g], r, sem).wait()
            pltpu.make_async_copy(edge_hbm.at[1, wid, g], c_, sem).wait()

        def fire_gather(rslab, b):
            pltpu.async_copy(h_hbm.at[rslab.at[b]], bufs[b], gsem[b])

        def wait_gather(rslab, b):
            pltpu.make_async_copy(h_hbm.at[rslab.at[b]], bufs[b],
                                  gsem[b]).wait()

        def scatter(cslab, b):
            pltpu.sync_copy(bufs[b], acc_sh.at[cslab.at[b]], add=True)

        slab_load(0, rsb[0], csb[0], ssem)
        slab_wait(0, rsb[0], csb[0], ssem)
        for b in range(NBUF):
            fire_gather(rsb[0], b)
        slab_load(1, rsb[1], csb[1], ssem)

        def do_group(g, pe, po, last):
            if not last:
                slab_wait(g + 1, rsb[po], csb[po], ssem)
            for b in range(NBUF):
                wait_gather(rsb[pe], b)
                scatter(csb[pe], b)
                if not last:
                    fire_gather(rsb[po], b)
            if not last:

                @pl.when(g + 2 < NGROUP)
                def _():
                    slab_load(g + 2, rsb[pe], csb[pe], ssem)

        def pair(p, carry):
            g = p * 2
            do_group(g, 0, 1, False)
            do_group(g + 1, 1, 0, False)
            return carry

        lax.fori_loop(0, NGROUP // 2 - 1, pair, 0)
        do_group(NGROUP - 2, 0, 1, False)
        do_group(NGROUP - 1, 1, 0, True)

        plsc.subcore_barrier()
        pltpu.sync_copy(acc_sh.at[pl.ds(row0, RPT)],
                        out_hbm.at[cid, pl.ds(row0, RPT)])

    return k(h, edge5d)


def _tc_epilogue(accp, degp, b2, pa2):
    BR = 2000

    def body(a_ref, deg_ref, b_ref, pa_ref, o_ref):
        s = a_ref[0] + a_ref[1]
        degb = deg_ref[...]
        deg = degb[:, 0:1] + degb[:, 1:2]
        dinv = jnp.where(deg > 0, lax.rsqrt(deg), 0.0)
        v = s * dinv + b_ref[...]
        pa = pa_ref[0, 0]
        o_ref[...] = jnp.where(v >= 0, v, pa * v)

    return pl.pallas_call(
        body,
        grid=(N // BR,),
        in_specs=[
            pl.BlockSpec((2, BR, D), lambda i: (0, i, 0)),
            pl.BlockSpec((BR, 2), lambda i: (i, 0)),
            pl.BlockSpec((1, D), lambda i: (0, 0)),
            pl.BlockSpec((1, 1), lambda i: (0, 0)),
        ],
        out_specs=pl.BlockSpec((BR, D), lambda i: (i, 0)),
        out_shape=jax.ShapeDtypeStruct((N, D), jnp.float32),
    )(accp, degp, b2, pa2)


def kernel(x, edge_index, W, b, prelu_a):
    edge5d = edge_index.reshape(2, NW, NGROUP, NBUF, CHUNK)
    edge4d = edge_index.reshape(2, NW, NCHUNK, CHUNK)
    degp = _sc_degree(edge4d)
    degp_t = degp.T
    h = _tc_linear(x, W.T, degp_t)
    accp = _sc_gather_scatter(h, edge5d)
    out = _tc_epilogue(accp, degp_t,
                       b.reshape(1, D), prelu_a.reshape(1, 1))
    return out


# speedup vs baseline: 1.0636x; 1.0636x over previous
import functools

import jax
import jax.numpy as jnp
from jax import lax
from jax.experimental import pallas as pl
from jax.experimental.pallas import tpu as pltpu
from jax.experimental.pallas import tpu_sc as plsc

N = 10000
D = 128
E = 320000

NC = 2
NS = 16
NW = NC * NS
EPW = E // NW
CHUNK = 80
NCHUNK = EPW // CHUNK
NBUF = 3
NGFULL = NCHUNK // NBUF
NTAIL = NCHUNK - NGFULL * NBUF
SLAB = NBUF * CHUNK

NP_ROWS = 10240
RPT = NP_ROWS // NS

DEG_FIRE = 25


def _sc_degree(col1d):
    mesh = plsc.VectorSubcoreMesh(core_axis_name="c", subcore_axis_name="s")

    @functools.partial(
        pl.kernel,
        out_type=jax.ShapeDtypeStruct((NC, NP_ROWS), jnp.float32),
        mesh=mesh,
        scratch_types=[
            pltpu.VMEM((EPW,), jnp.int32),
            pltpu.VMEM((CHUNK,), jnp.float32),
            pltpu.VMEM((RPT,), jnp.float32),
            pltpu.VMEM_SHARED((NP_ROWS,), jnp.float32),
            pltpu.SemaphoreType.DMA,
        ],
    )
    def k(col_hbm, deg_out, idx_v, ones_v, zero_v, deg_sh, sem):
        cid = lax.axis_index("c")
        sid = lax.axis_index("s")
        wid = sid * NC + cid
        pltpu.sync_copy(col_hbm.at[pl.ds(wid * EPW, EPW)], idx_v)

        for j in range(CHUNK // 16):
            ones_v[pl.ds(j * 16, 16)] = jnp.ones((16,), jnp.float32)

        def zfill(i, carry):
            zero_v[pl.ds(i * 16, 16)] = jnp.zeros((16,), jnp.float32)
            return carry

        lax.fori_loop(0, RPT // 16, zfill, 0)
        pltpu.sync_copy(zero_v, deg_sh.at[pl.ds(sid * RPT, RPT)])
        plsc.subcore_barrier()

        def body(grp, carry):
            for j in range(DEG_FIRE):
                c = grp * DEG_FIRE + j
                pltpu.async_copy(
                    ones_v, deg_sh.at[idx_v.at[pl.ds(c * CHUNK, CHUNK)]],
                    sem, add=True)
            for j in range(DEG_FIRE):
                c = grp * DEG_FIRE + j
                pltpu.make_async_copy(
                    ones_v, deg_sh.at[idx_v.at[pl.ds(c * CHUNK, CHUNK)]],
                    sem).wait()
            return carry

        lax.fori_loop(0, NCHUNK // DEG_FIRE, body, 0)
        plsc.subcore_barrier()
        pltpu.sync_copy(deg_sh.at[pl.ds(sid * RPT, RPT)],
                        deg_out.at[cid, pl.ds(sid * RPT, RPT)])

    return k(col1d)


def _tc_linear(x, wt, degp):
    BR = 2000

    def body(x_ref, wt_ref, deg_ref, o_ref):
        degb = deg_ref[...]
        deg = degb[:, 0:1] + degb[:, 1:2]
        dinv = jnp.where(deg > 0, lax.rsqrt(deg), 0.0)
        o_ref[...] = jnp.dot(x_ref[...] * dinv, wt_ref[...],
                             preferred_element_type=jnp.float32)

    return pl.pallas_call(
        body,
        grid=(N // BR,),
        in_specs=[
            pl.BlockSpec((BR, D), lambda i: (i, 0)),
            pl.BlockSpec((D, D), lambda i: (0, 0)),
            pl.BlockSpec((BR, 2), lambda i: (i, 0)),
        ],
        out_specs=pl.BlockSpec((BR, D), lambda i: (i, 0)),
        out_shape=jax.ShapeDtypeStruct((N, D), jnp.float32),
    )(x, wt, degp)


def _sc_gather_scatter(h, row1d, col1d):
    mesh = plsc.VectorSubcoreMesh(core_axis_name="c", subcore_axis_name="s")

    @functools.partial(
        pl.kernel,
        out_type=jax.ShapeDtypeStruct((NC, NP_ROWS, D), jnp.float32),
        mesh=mesh,
        scratch_types=[
            [pltpu.VMEM((CHUNK, D), jnp.float32) for _ in range(NBUF)],
            [pltpu.SemaphoreType.DMA for _ in range(NBUF)],
            [pltpu.VMEM((SLAB,), jnp.int32) for _ in range(2)],
            [pltpu.VMEM((SLAB,), jnp.int32) for _ in range(2)],
            pltpu.VMEM((NTAIL * CHUNK,), jnp.int32),
            pltpu.VMEM((NTAIL * CHUNK,), jnp.int32),
            pltpu.SemaphoreType.DMA,
            pltpu.VMEM_SHARED((NP_ROWS, D), jnp.float32),
        ],
    )
    def k(h_hbm, row_hbm, col_hbm, out_hbm, bufs, gsem, rsb, csb, trow,
          tcol, ssem, acc_sh):
        cid = lax.axis_index("c")
        sid = lax.axis_index("s")
        wid = sid * NC + cid
        wbase = wid * EPW
        row0 = sid * RPT

        def zfill(i, carry):
            for j in range(D // 16):
                bufs[0][i, pl.ds(j * 16, 16)] = jnp.zeros((16,), jnp.float32)
            return carry

        lax.fori_loop(0, CHUNK, zfill, 0)
        for j in range(RPT // CHUNK):
            pltpu.sync_copy(bufs[0], acc_sh.at[pl.ds(row0 + j * CHUNK, CHUNK)])
        plsc.subcore_barrier()

        def slab_load(g, r, c_, sem):
            pltpu.async_copy(row_hbm.at[pl.ds(wbase + g * SLAB, SLAB)], r,
                             sem)
            pltpu.async_copy(col_hbm.at[pl.ds(wbase + g * SLAB, SLAB)], c_,
                             sem)

        def slab_wait(g, r, c_, sem):
            pltpu.make_async_copy(row_hbm.at[pl.ds(wbase + g * SLAB, SLAB)],
                                  r, sem).wait()
            pltpu.make_async_copy(col_hbm.at[pl.ds(wbase + g * SLAB, SLAB)],
                                  c_, sem).wait()

        def fire_gather(rslab, b):
            pltpu.async_copy(h_hbm.at[rslab.at[pl.ds(b * CHUNK, CHUNK)]],
                             bufs[b], gsem[b])

        def wait_gather(rslab, b):
            pltpu.make_async_copy(h_hbm.at[rslab.at[pl.ds(b * CHUNK, CHUNK)]],
                                  bufs[b], gsem[b]).wait()

        def scatter(cslab, b):
            pltpu.sync_copy(bufs[b],
                            acc_sh.at[cslab.at[pl.ds(b * CHUNK, CHUNK)]],
                            add=True)

        pltpu.sync_copy(row_hbm.at[pl.ds(wbase + NGFULL * SLAB,
                                         NTAIL * CHUNK)], trow)
        pltpu.sync_copy(col_hbm.at[pl.ds(wbase + NGFULL * SLAB,
                                         NTAIL * CHUNK)], tcol)
        slab_load(0, rsb[0], csb[0], ssem)
        slab_wait(0, rsb[0], csb[0], ssem)
        for b in range(NBUF):
            fire_gather(rsb[0], b)
        slab_load(1, rsb[1], csb[1], ssem)

        def do_group(g, pe, po, nnext, fire_slab):
            for b in range(NBUF):
                wait_gather(rsb[pe], b)
                scatter(csb[pe], b)
                if b < nnext:
                    fire_gather(rsb[po], b)
            if fire_slab:
                slab_load(g + 2, rsb[pe], csb[pe], ssem)

        def pair(p, carry):
            g = p * 2
            slab_wait(g + 1, rsb[1], csb[1], ssem)
            do_group(g, 0, 1, NBUF, True)
            slab_wait(g + 2, rsb[0], csb[0], ssem)
            do_group(g + 1, 1, 0, NBUF, True)
            return carry

        lax.fori_loop(0, 19, pair, 0)
        slab_wait(39, rsb[1], csb[1], ssem)
        do_group(38, 0, 1, NBUF, True)
        slab_wait(40, rsb[0], csb[0], ssem)
        do_group(39, 1, 0, NBUF, False)
        for b in range(NBUF):
            wait_gather(rsb[0], b)
            scatter(csb[0], b)
            if b < NTAIL:
                fire_gather(trow, b)
        for b in range(NTAIL):
            wait_gather(trow, b)
            scatter(tcol, b)

        plsc.subcore_barrier()
        pltpu.sync_copy(acc_sh.at[pl.ds(row0, RPT)],
                        out_hbm.at[cid, pl.ds(row0, RPT)])

    return k(h, row1d, col1d)


def _tc_epilogue(accp, degp, b2, pa2):
    BR = 2000

    def body(a_ref, deg_ref, b_ref, pa_ref, o_ref):
        s = a_ref[0] + a_ref[1]
        degb = deg_ref[...]
        deg = degb[:, 0:1] + degb[:, 1:2]
        dinv = jnp.where(deg > 0, lax.rsqrt(deg), 0.0)
        v = s * dinv + b_ref[...]
        pa = pa_ref[0, 0]
        o_ref[...] = jnp.where(v >= 0, v, pa * v)

    return pl.pallas_call(
        body,
        grid=(N // BR,),
        in_specs=[
            pl.BlockSpec((2, BR, D), lambda i: (0, i, 0)),
            pl.BlockSpec((BR, 2), lambda i: (i, 0)),
            pl.BlockSpec((1, D), lambda i: (0, 0)),
            pl.BlockSpec((1, 1), lambda i: (0, 0)),
        ],
        out_specs=pl.BlockSpec((BR, D), lambda i: (i, 0)),
        out_shape=jax.ShapeDtypeStruct((N, D), jnp.float32),
    )(accp, degp, b2, pa2)


def kernel(x, edge_index, W, b, prelu_a):
    row1d = edge_index[0]
    col1d = edge_index[1]
    degp = _sc_degree(col1d)
    degp_t = degp.T
    h = _tc_linear(x, W.T, degp_t)
    accp = _sc_gather_scatter(h, row1d, col1d)
    out = _tc_epilogue(accp, degp_t,
                       b.reshape(1, D), prelu_a.reshape(1, 1))
    return out


# speedup vs baseline: 1.1569x; 1.0878x over previous
import functools

import jax
import jax.numpy as jnp
from jax import lax
from jax.experimental import pallas as pl
from jax.experimental.pallas import tpu as pltpu
from jax.experimental.pallas import tpu_sc as plsc

N = 10000
D = 128
E = 320000

NC = 2
NS = 16
NW = NC * NS
CHUNK = 128
TOTCH = E // CHUNK
BASECH = TOTCH // NW
NEXTRA = TOTCH - BASECH * NW
CHW_MAX = BASECH + 1
NSLAB = 3
NBUF = 2

NP_ROWS = 10240
RPT = NP_ROWS // NS

DEG_FIRE = 26


def _worker_chunks(wid):
    extra = jnp.minimum(wid, NEXTRA)
    cstart = wid * BASECH + extra
    return cstart


def _sc_degree(edge_index):
    mesh = plsc.VectorSubcoreMesh(core_axis_name="c", subcore_axis_name="s")

    @functools.partial(
        pl.kernel,
        out_type=jax.ShapeDtypeStruct((NC, NP_ROWS), jnp.float32),
        mesh=mesh,
        scratch_types=[
            pltpu.VMEM((2, CHW_MAX * CHUNK), jnp.int32),
            pltpu.VMEM((CHW_MAX * CHUNK,), jnp.int32),
            pltpu.VMEM((CHUNK,), jnp.float32),
            pltpu.VMEM((RPT,), jnp.float32),
            pltpu.VMEM_SHARED((NP_ROWS,), jnp.float32),
            pltpu.SemaphoreType.DMA,
        ],
    )
    def k(edge_hbm, deg_out, stg_v, idx_v, ones_v, zero_v, deg_sh, sem):
        cid = lax.axis_index("c")
        sid = lax.axis_index("s")
        wid = sid * NC + cid
        cstart = _worker_chunks(wid)

        @pl.when(wid < NEXTRA)
        def _():
            pltpu.sync_copy(
                edge_hbm.at[:, pl.ds(cstart * CHUNK, CHW_MAX * CHUNK)], stg_v)

        @pl.when(wid >= NEXTRA)
        def _():
            pltpu.sync_copy(
                edge_hbm.at[:, pl.ds(cstart * CHUNK, BASECH * CHUNK)],
                stg_v.at[:, pl.ds(0, BASECH * CHUNK)])

        def ccopy(i, carry):
            idx_v[pl.ds(i * 16, 16)] = stg_v[1, pl.ds(i * 16, 16)]
            return carry

        lax.fori_loop(0, CHW_MAX * CHUNK // 16, ccopy, 0)

        for j in range(CHUNK // 16):
            ones_v[pl.ds(j * 16, 16)] = jnp.ones((16,), jnp.float32)

        def zfill(i, carry):
            zero_v[pl.ds(i * 16, 16)] = jnp.zeros((16,), jnp.float32)
            return carry

        lax.fori_loop(0, RPT // 16, zfill, 0)
        pltpu.sync_copy(zero_v, deg_sh.at[pl.ds(sid * RPT, RPT)])
        plsc.subcore_barrier()

        def body(grp, carry):
            for j in range(DEG_FIRE):
                c = grp * DEG_FIRE + j
                pltpu.async_copy(
                    ones_v, deg_sh.at[idx_v.at[pl.ds(c * CHUNK, CHUNK)]],
                    sem, add=True)
            for j in range(DEG_FIRE):
                c = grp * DEG_FIRE + j
                pltpu.make_async_copy(
                    ones_v, deg_sh.at[idx_v.at[pl.ds(c * CHUNK, CHUNK)]],
                    sem).wait()
            return carry

        lax.fori_loop(0, BASECH // DEG_FIRE, body, 0)

        @pl.when(wid < NEXTRA)
        def _():
            pltpu.sync_copy(
                ones_v,
                deg_sh.at[idx_v.at[pl.ds(BASECH * CHUNK, CHUNK)]], add=True)

        plsc.subcore_barrier()
        pltpu.sync_copy(deg_sh.at[pl.ds(sid * RPT, RPT)],
                        deg_out.at[cid, pl.ds(sid * RPT, RPT)])

    return k(edge_index)


def _tc_linear(x, wt, degp):
    BR = 2000

    def body(x_ref, wt_ref, deg_ref, o_ref):
        degb = deg_ref[...]
        deg = degb[:, 0:1] + degb[:, 1:2]
        dinv = jnp.where(deg > 0, lax.rsqrt(deg), 0.0)
        o_ref[...] = jnp.dot(x_ref[...] * dinv, wt_ref[...],
                             preferred_element_type=jnp.float32)

    return pl.pallas_call(
        body,
        grid=(N // BR,),
        in_specs=[
            pl.BlockSpec((BR, D), lambda i: (i, 0)),
            pl.BlockSpec((D, D), lambda i: (0, 0)),
            pl.BlockSpec((BR, 2), lambda i: (i, 0)),
        ],
        out_specs=pl.BlockSpec((BR, D), lambda i: (i, 0)),
        out_shape=jax.ShapeDtypeStruct((N, D), jnp.float32),
    )(x, wt, degp)


def _sc_gather_scatter(h, edge_index):
    mesh = plsc.VectorSubcoreMesh(core_axis_name="c", subcore_axis_name="s")

    @functools.partial(
        pl.kernel,
        out_type=jax.ShapeDtypeStruct((NC, NP_ROWS, D), jnp.float32),
        mesh=mesh,
        scratch_types=[
            [pltpu.VMEM((CHUNK, D), jnp.float32) for _ in range(NBUF)],
            [pltpu.SemaphoreType.DMA for _ in range(NBUF)],
            [pltpu.VMEM((2, CHUNK), jnp.int32) for _ in range(NSLAB)],
            [pltpu.SemaphoreType.DMA for _ in range(NSLAB)],
            pltpu.VMEM_SHARED((NP_ROWS, D), jnp.float32),
        ],
    )
    def k(h_hbm, edge_hbm, out_hbm, bufs, gsem, slabs, ssem, acc_sh):
        cid = lax.axis_index("c")
        sid = lax.axis_index("s")
        wid = sid * NC + cid
        cstart = _worker_chunks(wid)
        nch = BASECH + jnp.where(wid < NEXTRA, 1, 0)
        row0 = sid * RPT

        def zfill(i, carry):
            for j in range(D // 16):
                bufs[0][i, pl.ds(j * 16, 16)] = jnp.zeros((16,), jnp.float32)
            return carry

        lax.fori_loop(0, CHUNK, zfill, 0)
        for j in range(RPT // CHUNK):
            pltpu.sync_copy(bufs[0], acc_sh.at[pl.ds(row0 + j * CHUNK, CHUNK)])
        plsc.subcore_barrier()

        def slab_src(c):
            return edge_hbm.at[:, pl.ds((cstart + c) * CHUNK, CHUNK)]

        def fire_slab(c, s):
            pltpu.async_copy(slab_src(c), slabs[s], ssem[s])

        def wait_slab(c, s):
            pltpu.make_async_copy(slab_src(c), slabs[s], ssem[s]).wait()

        def fire_gather(s, b):
            pltpu.async_copy(h_hbm.at[slabs[s].at[0]], bufs[b], gsem[b])

        def wait_gather(s, b):
            pltpu.make_async_copy(h_hbm.at[slabs[s].at[0]], bufs[b],
                                  gsem[b]).wait()

        def scatter(s, b):
            pltpu.sync_copy(bufs[b], acc_sh.at[slabs[s].at[1]], add=True)

        fire_slab(0, 0)
        fire_slab(1, 1)
        wait_slab(0, 0)
        fire_gather(0, 0)

        def step(c, carry):
            for s in range(NSLAB):
                @pl.when(c % NSLAB == s)
                def _():
                    sn = (s + 1) % NSLAB
                    sp = (s + 2) % NSLAB

                    @pl.when(c + 2 < nch)
                    def _():
                        fire_slab(c + 2, sp)

                    @pl.when(c + 1 < nch)
                    def _():
                        wait_slab(c + 1, sn)
                        for b in range(NBUF):
                            @pl.when((c + 1) % NBUF == b)
                            def _():
                                fire_gather(sn, b)

                    for b in range(NBUF):
                        @pl.when(c % NBUF == b)
                        def _():
                            wait_gather(s, b)
                            scatter(s, b)
            return carry

        lax.fori_loop(0, nch, step, 0)

        plsc.subcore_barrier()
        pltpu.sync_copy(acc_sh.at[pl.ds(row0, RPT)],
                        out_hbm.at[cid, pl.ds(row0, RPT)])

    return k(h, edge_index)


def _tc_epilogue(accp, degp, b2, pa2):
    BR = 2000

    def body(a_ref, deg_ref, b_ref, pa_ref, o_ref):
        s = a_ref[0] + a_ref[1]
        degb = deg_ref[...]
        deg = degb[:, 0:1] + degb[:, 1:2]
        dinv = jnp.where(deg > 0, lax.rsqrt(deg), 0.0)
        v = s * dinv + b_ref[...]
        pa = pa_ref[0, 0]
        o_ref[...] = jnp.where(v >= 0, v, pa * v)

    return pl.pallas_call(
        body,
        grid=(N // BR,),
        in_specs=[
            pl.BlockSpec((2, BR, D), lambda i: (0, i, 0)),
            pl.BlockSpec((BR, 2), lambda i: (i, 0)),
            pl.BlockSpec((1, D), lambda i: (0, 0)),
            pl.BlockSpec((1, 1), lambda i: (0, 0)),
        ],
        out_specs=pl.BlockSpec((BR, D), lambda i: (i, 0)),
        out_shape=jax.ShapeDtypeStruct((N, D), jnp.float32),
    )(accp, degp, b2, pa2)


def kernel(x, edge_index, W, b, prelu_a):
    degp = _sc_degree(edge_index)
    degp_t = degp.T
    h = _tc_linear(x, W.T, degp_t)
    accp = _sc_gather_scatter(h, edge_index)
    out = _tc_epilogue(accp, degp_t,
                       b.reshape(1, D), prelu_a.reshape(1, 1))
    return out


# speedup vs baseline: 1.1816x; 1.0213x over previous
import functools

import jax
import jax.numpy as jnp
from jax import lax
from jax.experimental import pallas as pl
from jax.experimental.pallas import tpu as pltpu
from jax.experimental.pallas import tpu_sc as plsc

N = 10000
D = 128
E = 320000

NC = 2
NS = 16
NW = NC * NS
CHUNK = 128
TOTCH = E // CHUNK
BASECH = TOTCH // NW
NEXTRA = TOTCH - BASECH * NW
CHW_MAX = BASECH + 1
NSLAB = 3
NBUF = 2

NP_ROWS = 10240
RPT = NP_ROWS // NS

DEG_FIRE = 26


def _worker_chunks(wid):
    extra = jnp.minimum(wid, NEXTRA)
    cstart = wid * BASECH + extra
    return cstart


def _sc_degree(edge_index):
    mesh = plsc.VectorSubcoreMesh(core_axis_name="c", subcore_axis_name="s")

    @functools.partial(
        pl.kernel,
        out_type=jax.ShapeDtypeStruct((NC, NP_ROWS), jnp.float32),
        mesh=mesh,
        scratch_types=[
            pltpu.VMEM((2, CHW_MAX * CHUNK), jnp.int32),
            pltpu.VMEM((CHUNK,), jnp.float32),
            pltpu.VMEM((RPT,), jnp.float32),
            pltpu.VMEM_SHARED((NP_ROWS,), jnp.float32),
            pltpu.SemaphoreType.DMA,
        ],
    )
    def k(edge_hbm, deg_out, stg_v, ones_v, zero_v, deg_sh, sem):
        cid = lax.axis_index("c")
        sid = lax.axis_index("s")
        wid = sid * NC + cid
        cstart = _worker_chunks(wid)

        @pl.when(wid < NEXTRA)
        def _():
            pltpu.sync_copy(
                edge_hbm.at[:, pl.ds(cstart * CHUNK, CHW_MAX * CHUNK)], stg_v)

        @pl.when(wid >= NEXTRA)
        def _():
            pltpu.sync_copy(
                edge_hbm.at[:, pl.ds(cstart * CHUNK, BASECH * CHUNK)],
                stg_v.at[:, pl.ds(0, BASECH * CHUNK)])

        for j in range(CHUNK // 16):
            ones_v[pl.ds(j * 16, 16)] = jnp.ones((16,), jnp.float32)

        def zfill(i, carry):
            zero_v[pl.ds(i * 16, 16)] = jnp.zeros((16,), jnp.float32)
            return carry

        lax.fori_loop(0, RPT // 16, zfill, 0)
        pltpu.sync_copy(zero_v, deg_sh.at[pl.ds(sid * RPT, RPT)])
        plsc.subcore_barrier()

        def body(grp, carry):
            for j in range(DEG_FIRE):
                c = grp * DEG_FIRE + j
                pltpu.async_copy(
                    ones_v,
                    deg_sh.at[stg_v.at[1, pl.ds(c * CHUNK, CHUNK)]],
                    sem, add=True)
            for j in range(DEG_FIRE):
                c = grp * DEG_FIRE + j
                pltpu.make_async_copy(
                    ones_v,
                    deg_sh.at[stg_v.at[1, pl.ds(c * CHUNK, CHUNK)]],
                    sem).wait()
            return carry

        lax.fori_loop(0, BASECH // DEG_FIRE, body, 0)

        @pl.when(wid < NEXTRA)
        def _():
            pltpu.sync_copy(
                ones_v,
                deg_sh.at[stg_v.at[1, pl.ds(BASECH * CHUNK, CHUNK)]],
                add=True)

        plsc.subcore_barrier()
        pltpu.sync_copy(deg_sh.at[pl.ds(sid * RPT, RPT)],
                        deg_out.at[cid, pl.ds(sid * RPT, RPT)])

    return k(edge_index)


def _tc_linear(x, wt, degp):
    BR = 2000

    def body(x_ref, wt_ref, deg_ref, o_ref):
        degb = deg_ref[...]
        deg = degb[:, 0:1] + degb[:, 1:2]
        dinv = jnp.where(deg > 0, lax.rsqrt(deg), 0.0)
        o_ref[...] = jnp.dot(x_ref[...] * dinv, wt_ref[...],
                             preferred_element_type=jnp.float32)

    return pl.pallas_call(
        body,
        grid=(N // BR,),
        in_specs=[
            pl.BlockSpec((BR, D), lambda i: (i, 0)),
            pl.BlockSpec((D, D), lambda i: (0, 0)),
            pl.BlockSpec((BR, 2), lambda i: (i, 0)),
        ],
        out_specs=pl.BlockSpec((BR, D), lambda i: (i, 0)),
        out_shape=jax.ShapeDtypeStruct((N, D), jnp.float32),
    )(x, wt, degp)


def _sc_gather_scatter(h, edge_index):
    mesh = plsc.VectorSubcoreMesh(core_axis_name="c", subcore_axis_name="s")

    @functools.partial(
        pl.kernel,
        out_type=jax.ShapeDtypeStruct((NC, NP_ROWS, D), jnp.float32),
        mesh=mesh,
        scratch_types=[
            [pltpu.VMEM((CHUNK, D), jnp.float32) for _ in range(NBUF)],
            [pltpu.SemaphoreType.DMA for _ in range(NBUF)],
            [pltpu.VMEM((2, CHUNK), jnp.int32) for _ in range(NSLAB)],
            [pltpu.SemaphoreType.DMA for _ in range(NSLAB)],
            pltpu.VMEM_SHARED((NP_ROWS, D), jnp.float32),
        ],
    )
    def k(h_hbm, edge_hbm, out_hbm, bufs, gsem, slabs, ssem, acc_sh):
        cid = lax.axis_index("c")
        sid = lax.axis_index("s")
        wid = sid * NC + cid
        cstart = _worker_chunks(wid)
        nch = BASECH + jnp.where(wid < NEXTRA, 1, 0)
        row0 = sid * RPT

        def zfill(i, carry):
            for j in range(D // 16):
                bufs[0][i, pl.ds(j * 16, 16)] = jnp.zeros((16,), jnp.float32)
            return carry

        lax.fori_loop(0, CHUNK, zfill, 0)
        for j in range(RPT // CHUNK):
            pltpu.sync_copy(bufs[0], acc_sh.at[pl.ds(row0 + j * CHUNK, CHUNK)])
        plsc.subcore_barrier()

        def slab_src(c):
            return edge_hbm.at[:, pl.ds((cstart + c) * CHUNK, CHUNK)]

        def fire_slab(c, s):
            pltpu.async_copy(slab_src(c), slabs[s], ssem[s])

        def wait_slab(c, s):
            pltpu.make_async_copy(slab_src(c), slabs[s], ssem[s]).wait()

        def fire_gather(s, b):
            pltpu.async_copy(h_hbm.at[slabs[s].at[0]], bufs[b], gsem[b])

        def wait_gather(s, b):
            pltpu.make_async_copy(h_hbm.at[slabs[s].at[0]], bufs[b],
                                  gsem[b]).wait()

        def scatter(s, b):
            pltpu.sync_copy(bufs[b], acc_sh.at[slabs[s].at[1]], add=True)

        fire_slab(0, 0)
        fire_slab(1, 1)
        wait_slab(0, 0)
        fire_gather(0, 0)

        def step(c, carry):
            for s in range(NSLAB):
                @pl.when(c % NSLAB == s)
                def _():
                    sn = (s + 1) % NSLAB
                    sp = (s + 2) % NSLAB

                    @pl.when(c + 2 < nch)
                    def _():
                        fire_slab(c + 2, sp)

                    @pl.when(c + 1 < nch)
                    def _():
                        wait_slab(c + 1, sn)
                        for b in range(NBUF):
                            @pl.when((c + 1) % NBUF == b)
                            def _():
                                fire_gather(sn, b)

                    for b in range(NBUF):
                        @pl.when(c % NBUF == b)
                        def _():
                            wait_gather(s, b)
                            scatter(s, b)
            return carry

        lax.fori_loop(0, nch, step, 0)

        plsc.subcore_barrier()
        pltpu.sync_copy(acc_sh.at[pl.ds(row0, RPT)],
                        out_hbm.at[cid, pl.ds(row0, RPT)])

    return k(h, edge_index)


def _tc_epilogue(accp, degp, b2, pa2):
    BR = 2000

    def body(a_ref, deg_ref, b_ref, pa_ref, o_ref):
        s = a_ref[0] + a_ref[1]
        degb = deg_ref[...]
        deg = degb[:, 0:1] + degb[:, 1:2]
        dinv = jnp.where(deg > 0, lax.rsqrt(deg), 0.0)
        v = s * dinv + b_ref[...]
        pa = pa_ref[0, 0]
        o_ref[...] = jnp.where(v >= 0, v, pa * v)

    return pl.pallas_call(
        body,
        grid=(N // BR,),
        in_specs=[
            pl.BlockSpec((2, BR, D), lambda i: (0, i, 0)),
            pl.BlockSpec((BR, 2), lambda i: (i, 0)),
            pl.BlockSpec((1, D), lambda i: (0, 0)),
            pl.BlockSpec((1, 1), lambda i: (0, 0)),
        ],
        out_specs=pl.BlockSpec((BR, D), lambda i: (i, 0)),
        out_shape=jax.ShapeDtypeStruct((N, D), jnp.float32),
    )(accp, degp, b2, pa2)


def kernel(x, edge_index, W, b, prelu_a):
    degp = _sc_degree(edge_index)
    degp_t = degp.T
    h = _tc_linear(x, W.T, degp_t)
    accp = _sc_gather_scatter(h, edge_index)
    out = _tc_epilogue(accp, degp_t,
                       b.reshape(1, D), prelu_a.reshape(1, 1))
    return out
